# baseline (speedup 1.0000x reference)
import functools

import jax
import jax.numpy as jnp
from jax import lax
from jax.experimental import pallas as pl
from jax.experimental.pallas import tpu as pltpu
from jax.experimental.pallas import tpu_sc as plsc

EMB = 32
PACK = 128 // EMB
NUM_CORES = 2
NUM_SUBCORES = 16
LANES = 16
NUM_WORKERS = NUM_CORES * NUM_SUBCORES
CHUNK = 256


def _mf_sc(user_id, game_id, user_table4, game_table4):
    batch = user_id.shape[0]
    bpw = batch // NUM_WORKERS
    mesh = plsc.VectorSubcoreMesh(core_axis_name="c", subcore_axis_name="s")

    @functools.partial(
        pl.kernel,
        mesh=mesh,
        out_type=jax.ShapeDtypeStruct((batch,), jnp.float32),
        compiler_params=pltpu.CompilerParams(needs_layout_passes=False),
        scratch_types=[
            pltpu.VMEM((bpw,), jnp.int32),
            pltpu.VMEM((bpw,), jnp.int32),
            pltpu.VMEM((bpw,), jnp.int32),
            pltpu.VMEM((bpw,), jnp.int32),
            pltpu.VMEM((CHUNK, 128), jnp.float32),
            pltpu.VMEM((CHUNK, 128), jnp.float32),
            pltpu.VMEM((bpw,), jnp.float32),
            pltpu.SemaphoreType.DMA,
            pltpu.SemaphoreType.DMA,
        ],
    )
    def mf_kernel(uid_hbm, gid_hbm, ut_hbm, gt_hbm, out_hbm,
                  uid_v, gid_v, ub_v, gb_v, u_v, g_v, o_v, sem_u, sem_g):
        wid = lax.axis_index("s") * NUM_CORES + lax.axis_index("c")
        base = wid * bpw
        pltpu.sync_copy(uid_hbm.at[pl.ds(base, bpw)], uid_v)
        pltpu.sync_copy(gid_hbm.at[pl.ds(base, bpw)], gid_v)

        @pl.loop(0, bpw, step=LANES)
        def _(i0):
            s = pl.ds(i0, LANES)
            ub_v[s] = lax.shift_right_logical(uid_v[s], PACK.bit_length() - 1)
            gb_v[s] = lax.shift_right_logical(gid_v[s], PACK.bit_length() - 1)

        lanes = lax.iota(jnp.int32, LANES)

        for c in range(bpw // CHUNK):
            r0 = c * CHUNK
            cp_u = pltpu.async_copy(
                ut_hbm.at[ub_v.at[pl.ds(r0, CHUNK)]], u_v, sem_u)
            cp_g = pltpu.async_copy(
                gt_hbm.at[gb_v.at[pl.ds(r0, CHUNK)]], g_v, sem_g)
            cp_u.wait()
            cp_g.wait()

            @pl.loop(0, CHUNK, step=LANES)
            def _(k0):
                rows = k0 + lanes
                uoff = lax.shift_left(
                    lax.bitwise_and(uid_v[pl.ds(r0 + k0, LANES)],
                                    jnp.int32(PACK - 1)), 5)
                goff = lax.shift_left(
                    lax.bitwise_and(gid_v[pl.ds(r0 + k0, LANES)],
                                    jnp.int32(PACK - 1)), 5)
                acc = jnp.zeros((LANES,), jnp.float32)
                for j in range(EMB):
                    u_col = plsc.load_gather(u_v, [rows, uoff + j])
                    g_col = plsc.load_gather(g_v, [rows, goff + j])
                    acc = acc + u_col * g_col
                o_v[pl.ds(r0 + k0, LANES)] = 10.0 / (1.0 + jnp.exp(-acc))

        pltpu.sync_copy(o_v, out_hbm.at[pl.ds(base, bpw)])

    return mf_kernel(user_id, game_id, user_table4, game_table4)


def kernel(user_id, game_id, user_table, game_table):
    user_id = user_id.astype(jnp.int32)
    game_id = game_id.astype(jnp.int32)
    nu = user_table.shape[0] // PACK
    ng = game_table.shape[0] // PACK
    ut4 = user_table.reshape(nu, EMB * PACK)
    gt4 = game_table.reshape(ng, EMB * PACK)
    return _mf_sc(user_id, game_id, ut4, gt4)


# speedup vs baseline: 1.4901x; 1.4901x over previous
import functools

import jax
import jax.numpy as jnp
from jax import lax
from jax.experimental import pallas as pl
from jax.experimental.pallas import tpu as pltpu
from jax.experimental.pallas import tpu_sc as plsc

EMB = 32
NUM_CORES = 2
NUM_SUBCORES = 16
LANES = 16
NUM_WORKERS = NUM_CORES * NUM_SUBCORES
CHUNK = 128


def _mf_sc(user_id, game_id, user_table, game_table):
    batch = user_id.shape[0]
    bpw = batch // NUM_WORKERS
    mesh = plsc.VectorSubcoreMesh(core_axis_name="c", subcore_axis_name="s")

    @functools.partial(
        pl.kernel,
        mesh=mesh,
        out_type=jax.ShapeDtypeStruct((batch,), jnp.float32),
        compiler_params=pltpu.CompilerParams(needs_layout_passes=False),
        scratch_types=[
            pltpu.VMEM((bpw,), jnp.int32),
            pltpu.VMEM((bpw,), jnp.int32),
            pltpu.VMEM((CHUNK, EMB), jnp.float32),
            pltpu.VMEM((CHUNK, EMB), jnp.float32),
            pltpu.VMEM((bpw,), jnp.float32),
            pltpu.SemaphoreType.DMA,
            pltpu.SemaphoreType.DMA,
        ],
    )
    def mf_kernel(uid_hbm, gid_hbm, ut_hbm, gt_hbm, out_hbm,
                  uid_v, gid_v, u_v, g_v, o_v, sem_u, sem_g):
        wid = lax.axis_index("s") * NUM_CORES + lax.axis_index("c")
        base = wid * bpw
        pltpu.sync_copy(uid_hbm.at[pl.ds(base, bpw)], uid_v)
        pltpu.sync_copy(gid_hbm.at[pl.ds(base, bpw)], gid_v)

        lanes = lax.iota(jnp.int32, LANES)

        @pl.loop(0, bpw, step=CHUNK)
        def _(r0):
            copies = []
            for k0 in range(0, CHUNK, LANES):
                uvec = uid_v[pl.ds(r0 + k0, LANES)]
                gvec = gid_v[pl.ds(r0 + k0, LANES)]
                for j in range(LANES):
                    copies.append(pltpu.async_copy(
                        ut_hbm.at[pl.ds(uvec[j], 1)],
                        u_v.at[pl.ds(k0 + j, 1)], sem_u))
                    copies.append(pltpu.async_copy(
                        gt_hbm.at[pl.ds(gvec[j], 1)],
                        g_v.at[pl.ds(k0 + j, 1)], sem_g))
            for cp in copies:
                cp.wait()

            @pl.loop(0, CHUNK, step=LANES)
            def _(k0):
                rows = k0 + lanes
                acc = jnp.zeros((LANES,), jnp.float32)
                for j in range(EMB):
                    cols = jnp.full((LANES,), j, jnp.int32)
                    u_col = plsc.load_gather(u_v, [rows, cols])
                    g_col = plsc.load_gather(g_v, [rows, cols])
                    acc = acc + u_col * g_col
                o_v[pl.ds(r0 + k0, LANES)] = 10.0 / (1.0 + jnp.exp(-acc))

        pltpu.sync_copy(o_v, out_hbm.at[pl.ds(base, bpw)])

    return mf_kernel(user_id, game_id, user_table, game_table)


def kernel(user_id, game_id, user_table, game_table):
    user_id = user_id.astype(jnp.int32)
    game_id = game_id.astype(jnp.int32)
    return _mf_sc(user_id, game_id, user_table, game_table)


# speedup vs baseline: 1.4905x; 1.0003x over previous
import functools

import jax
import jax.numpy as jnp
from jax import lax
from jax.experimental import pallas as pl
from jax.experimental.pallas import tpu as pltpu
from jax.experimental.pallas import tpu_sc as plsc

EMB = 32
NUM_CORES = 2
NUM_SUBCORES = 16
LANES = 16
NUM_WORKERS = NUM_CORES * NUM_SUBCORES
CHUNK = 128


def _mf_sc(user_id, game_id, user_table, game_table):
    batch = user_id.shape[0]
    bpw = batch // NUM_WORKERS
    mesh = plsc.VectorSubcoreMesh(core_axis_name="c", subcore_axis_name="s")

    @functools.partial(
        pl.kernel,
        mesh=mesh,
        out_type=jax.ShapeDtypeStruct((batch,), jnp.float32),
        compiler_params=pltpu.CompilerParams(
            needs_layout_passes=False, skip_device_barrier=True),
        scratch_types=[
            pltpu.VMEM((bpw,), jnp.int32),
            pltpu.VMEM((bpw,), jnp.int32),
            pltpu.VMEM((CHUNK, EMB), jnp.float32),
            pltpu.VMEM((CHUNK, EMB), jnp.float32),
            pltpu.VMEM((bpw,), jnp.float32),
            pltpu.SemaphoreType.DMA,
            pltpu.SemaphoreType.DMA,
        ],
    )
    def mf_kernel(uid_hbm, gid_hbm, ut_hbm, gt_hbm, out_hbm,
                  uid_v, gid_v, u_v, g_v, o_v, sem_u, sem_g):
        wid = lax.axis_index("s") * NUM_CORES + lax.axis_index("c")
        base = wid * bpw
        pltpu.sync_copy(uid_hbm.at[pl.ds(base, bpw)], uid_v)
        pltpu.sync_copy(gid_hbm.at[pl.ds(base, bpw)], gid_v)

        lanes = lax.iota(jnp.int32, LANES)

        @pl.loop(0, bpw, step=CHUNK)
        def _(r0):
            copies = []
            for k0 in range(0, CHUNK, LANES):
                uvec = uid_v[pl.ds(r0 + k0, LANES)]
                gvec = gid_v[pl.ds(r0 + k0, LANES)]
                for j in range(LANES):
                    copies.append(pltpu.async_copy(
                        ut_hbm.at[pl.ds(uvec[j], 1)],
                        u_v.at[pl.ds(k0 + j, 1)], sem_u))
                    copies.append(pltpu.async_copy(
                        gt_hbm.at[pl.ds(gvec[j], 1)],
                        g_v.at[pl.ds(k0 + j, 1)], sem_g))
            for cp in copies:
                cp.wait()

            @pl.loop(0, CHUNK, step=LANES)
            def _(k0):
                rows = k0 + lanes
                acc = jnp.zeros((LANES,), jnp.float32)
                for j in range(EMB):
                    cols = jnp.full((LANES,), j, jnp.int32)
                    u_col = plsc.load_gather(u_v, [rows, cols])
                    g_col = plsc.load_gather(g_v, [rows, cols])
                    acc = acc + u_col * g_col
                o_v[pl.ds(r0 + k0, LANES)] = 10.0 / (1.0 + jnp.exp(-acc))

        pltpu.sync_copy(o_v, out_hbm.at[pl.ds(base, bpw)])

    return mf_kernel(user_id, game_id, user_table, game_table)


def kernel(user_id, game_id, user_table, game_table):
    user_id = user_id.astype(jnp.int32)
    game_id = game_id.astype(jnp.int32)
    return _mf_sc(user_id, game_id, user_table, game_table)
